# baseline (speedup 1.0000x reference)
import functools

import jax
import jax.numpy as jnp
from jax import lax
from jax.experimental import pallas as pl
from jax.experimental.pallas import tpu as pltpu
from jax.experimental.pallas import tpu_sc as plsc

NC = 2
NS = 16
NW = NC * NS
ROWS = 65536
COLS = 100
MCOLS = 10
ROWS_PER = ROWS // NW
A_PER = ROWS_PER * MCOLS
A_TOT = ROWS * MCOLS
CH = 512
NCH = ROWS_PER // CH


def _sc_partials(x2d, aflat):
    mesh = plsc.VectorSubcoreMesh(core_axis_name="c", subcore_axis_name="s")

    @functools.partial(
        pl.kernel,
        out_type=jax.ShapeDtypeStruct((NW, 16), jnp.float32),
        mesh=mesh,
        scratch_types=[
            pltpu.VMEM((CH * MCOLS + 16,), jnp.float32),
            pltpu.VMEM((CH, COLS), jnp.float32),
            pltpu.VMEM((16,), jnp.float32),
        ],
    )
    def k(x2d_hbm, aflat_hbm, out_hbm, a_v, b_v, res_v):
        wid = lax.axis_index("s") * NC + lax.axis_index("c")
        base_row = wid * ROWS_PER
        lane = lax.iota(jnp.int32, 16)
        mask = lane < MCOLS

        def chunk(c, acc):
            row0 = base_row + c * CH
            pltpu.sync_copy(x2d_hbm.at[pl.ds(row0, CH)], b_v)
            pltpu.sync_copy(aflat_hbm.at[pl.ds(row0 * MCOLS, CH * MCOLS)],
                            a_v.at[pl.ds(0, CH * MCOLS)])

            def body(q, acc2):
                va = a_v[pl.ds(q * MCOLS, 16)]
                vb = b_v[q, pl.ds(0, 16)]
                d = jnp.abs(va - vb)
                return acc2 + jnp.where(mask, d, 0.0)

            return lax.fori_loop(0, CH, body, acc)

        acc = lax.fori_loop(0, NCH, chunk, jnp.zeros((16,), jnp.float32))
        res_v[...] = acc
        pltpu.sync_copy(res_v, out_hbm.at[wid])

    return k(x2d, aflat)


def kernel(x):
    aflat = x.reshape(-1)[:A_TOT]
    partials = _sc_partials(x, aflat)
    return jnp.sum(partials)


# speedup vs baseline: 1.5474x; 1.5474x over previous
import functools

import jax
import jax.numpy as jnp
from jax import lax
from jax.experimental import pallas as pl
from jax.experimental.pallas import tpu as pltpu
from jax.experimental.pallas import tpu_sc as plsc

NC = 2
NS = 16
NW = NC * NS
ROWS = 65536
COLS = 100
MCOLS = 10
ROWS_PER = ROWS // NW
A_PER = ROWS_PER * MCOLS
A_TOT = ROWS * MCOLS
CH = 512
NCH = ROWS_PER // CH


def _sc_partials(x2d, aflat):
    mesh = plsc.VectorSubcoreMesh(core_axis_name="c", subcore_axis_name="s")

    @functools.partial(
        pl.kernel,
        out_type=jax.ShapeDtypeStruct((NW, 16), jnp.float32),
        mesh=mesh,
        scratch_types=[
            pltpu.VMEM((CH * MCOLS + 16,), jnp.float32),
            pltpu.VMEM((CH, COLS), jnp.float32),
            pltpu.VMEM((16,), jnp.float32),
        ],
    )
    def k(x2d_hbm, aflat_hbm, out_hbm, a_v, b_v, res_v):
        wid = lax.axis_index("s") * NC + lax.axis_index("c")
        base_row = wid * ROWS_PER
        lane = lax.iota(jnp.int32, 16)
        mask = lane < MCOLS

        def chunk(c, acc):
            row0 = base_row + c * CH
            pltpu.sync_copy(x2d_hbm.at[pl.ds(row0, CH)], b_v)
            pltpu.sync_copy(aflat_hbm.at[pl.ds(row0 * MCOLS, CH * MCOLS)],
                            a_v.at[pl.ds(0, CH * MCOLS)])

            def body(q, acc2):
                va = a_v[pl.ds(q * MCOLS, 16)]
                vb = b_v[q, pl.ds(0, 16)]
                d = jnp.abs(va - vb)
                return acc2 + jnp.where(mask, d, 0.0)

            return lax.fori_loop(0, CH, body, acc)

        acc = lax.fori_loop(0, NCH, chunk, jnp.zeros((16,), jnp.float32))
        res_v[...] = acc
        pltpu.sync_copy(res_v, out_hbm.at[wid])

    return k(x2d, aflat)


def kernel(x):
    aflat = x[:6560].reshape(-1)
    partials = _sc_partials(x, aflat)
    return jnp.sum(partials)
